# Initial kernel scaffold; baseline (speedup 1.0000x reference)
import functools
import math

import jax
import jax.numpy as jnp
from jax import lax
from jax.experimental import pallas as pl
from jax.experimental.pallas import tpu as pltpu
from jax.experimental.pallas import tpu_sc as plsc

B = 16
G = 64
P = 500
NC = 2
NS = 16
L = 16
NW = NC * NS
NBP = 8192
CH = NBP // NW

LN_ALPHA = math.log(1.0 - 0.01)

_mesh = plsc.VectorSubcoreMesh(core_axis_name="c", subcore_axis_name="s")


@functools.partial(
    pl.kernel,
    mesh=_mesh,
    out_type=jax.ShapeDtypeStruct((4, NBP), jnp.float32),
    scratch_types=[
        pltpu.VMEM((B * G * 4,), jnp.float32),
        pltpu.VMEM((B * 4,), jnp.float32),
        pltpu.VMEM((CH,), jnp.int32),
        pltpu.VMEM((CH,), jnp.int32),
        pltpu.VMEM((4, CH), jnp.float32),
        pltpu.VMEM((4, CH), jnp.float32),
    ],
)
def _noised_gt_sc(gt_hbm, sc_hbm, idx_hbm, t_hbm, nz_hbm, out_hbm,
                  gt_v, sc_v, idx_v, t_v, nz_v, o_v):
    wid = lax.axis_index("s") * NC + lax.axis_index("c")
    base = wid * CH
    pltpu.sync_copy(gt_hbm, gt_v)
    pltpu.sync_copy(sc_hbm, sc_v)
    pltpu.sync_copy(idx_hbm.at[pl.ds(base, CH)], idx_v)
    pltpu.sync_copy(t_hbm.at[pl.ds(base, CH)], t_v)
    for c in range(4):
        pltpu.sync_copy(nz_hbm.at[c, pl.ds(base, CH)], nz_v.at[c])
    for j in range(CH // L):
        sl = pl.ds(j * L, L)
        li = idx_v[sl]
        tf = t_v[sl].astype(jnp.float32)
        alpha = jnp.exp(tf * LN_ALPHA)
        sqrt_a = jnp.exp(tf * (0.5 * LN_ALPHA))
        x = 1.0 - alpha
        xi = plsc.bitcast(x, jnp.int32)
        y = plsc.bitcast((xi >> 1) + 0x1FBD1DF5, jnp.float32)
        y = 0.5 * (y + x / y)
        sqrt_b = 0.5 * (y + x / y)
        bp = base + j * L + lax.iota(jnp.int32, (16,))
        b = jnp.minimum(bp // P, B - 1)
        gbase = (b * G + li) * 4
        sbase = b * 4
        for c in range(4):
            box = plsc.load_gather(gt_v, [gbase + c])
            s = plsc.load_gather(sc_v, [sbase + c])
            o_v[c, sl] = s * (box * sqrt_a + nz_v[c, sl] * sqrt_b)
    for c in range(4):
        pltpu.sync_copy(o_v.at[c], out_hbm.at[c, pl.ds(base, CH)])


def kernel(gt_boxes, scales, sampled_indices, t, noise):
    gt_flat = gt_boxes.reshape(-1).astype(jnp.float32)
    sc_flat = scales.reshape(-1).astype(jnp.float32)
    idx = jnp.pad(sampled_indices.reshape(-1).astype(jnp.int32), (0, NBP - B * P))
    tt = jnp.pad(t.reshape(-1).astype(jnp.int32), (0, NBP - B * P))
    nz = jnp.pad(noise.reshape(B * P, 4), ((0, NBP - B * P), (0, 0))).T
    out = _noised_gt_sc(gt_flat, sc_flat, idx, tt, nz)
    return out.T[: B * P].reshape(B, P, 4)


# speedup vs baseline: 9.3998x; 9.3998x over previous
import functools
import math

import jax
import jax.numpy as jnp
from jax import lax
from jax.experimental import pallas as pl
from jax.experimental.pallas import tpu as pltpu
from jax.experimental.pallas import tpu_sc as plsc

B = 16
G = 64
P = 500
NC = 2
NS = 16
L = 16
NW = NC * NS
NBP = 8192
CH = NBP // NW

LN_ALPHA = math.log(1.0 - 0.01)

_mesh = plsc.VectorSubcoreMesh(core_axis_name="c", subcore_axis_name="s")


@functools.partial(
    pl.kernel,
    mesh=_mesh,
    compiler_params=pltpu.CompilerParams(needs_layout_passes=False),
    out_type=jax.ShapeDtypeStruct((4, NBP), jnp.float32),
    scratch_types=[
        pltpu.VMEM((B * G * 4,), jnp.float32),
        pltpu.VMEM((B * 4,), jnp.float32),
        pltpu.VMEM((CH,), jnp.int32),
        pltpu.VMEM((CH,), jnp.int32),
        pltpu.VMEM((CH,), jnp.int32),
        pltpu.VMEM((4, CH), jnp.float32),
        pltpu.VMEM((4, CH), jnp.float32),
    ],
)
def _noised_gt_sc(gt_hbm, sc_hbm, idx_hbm, t_hbm, bid_hbm, nz_hbm, out_hbm,
                  gt_v, sc_v, idx_v, t_v, bid_v, nz_v, o_v):
    wid = lax.axis_index("s") * NC + lax.axis_index("c")
    base = wid * CH
    pltpu.sync_copy(gt_hbm, gt_v)
    pltpu.sync_copy(sc_hbm, sc_v)
    pltpu.sync_copy(idx_hbm.at[pl.ds(base, CH)], idx_v)
    pltpu.sync_copy(t_hbm.at[pl.ds(base, CH)], t_v)
    pltpu.sync_copy(bid_hbm.at[pl.ds(base, CH)], bid_v)
    for c in range(4):
        pltpu.sync_copy(nz_hbm.at[c, pl.ds(base, CH)], nz_v.at[c])
    for j in range(CH // L):
        sl = pl.ds(j * L, L)
        li = idx_v[sl]
        tf = t_v[sl].astype(jnp.float32)
        alpha = jnp.exp(tf * LN_ALPHA)
        sqrt_a = jnp.exp(tf * (0.5 * LN_ALPHA))
        x = 1.0 - alpha
        y = jnp.where(x > 0.25, 0.71, jnp.where(x > 0.04, 0.275, 0.105))
        y = 0.5 * (y + x / y)
        y = 0.5 * (y + x / y)
        y = 0.5 * (y + x / y)
        y = 0.5 * (y + x / y)
        sqrt_b = jnp.where(x > 0.0, y, 0.0)
        b = bid_v[sl]
        gbase = (b * G + li) * 4
        sbase = b * 4
        for c in range(4):
            box = plsc.load_gather(gt_v, [gbase + c])
            s = plsc.load_gather(sc_v, [sbase + c])
            o_v[c, sl] = s * (box * sqrt_a + nz_v[c, sl] * sqrt_b)
    for c in range(4):
        pltpu.sync_copy(o_v.at[c], out_hbm.at[c, pl.ds(base, CH)])


def kernel(gt_boxes, scales, sampled_indices, t, noise):
    gt_flat = gt_boxes.reshape(-1).astype(jnp.float32)
    sc_flat = scales.reshape(-1).astype(jnp.float32)
    idx = jnp.pad(sampled_indices.reshape(-1).astype(jnp.int32), (0, NBP - B * P))
    tt = jnp.pad(t.reshape(-1).astype(jnp.int32), (0, NBP - B * P))
    nz = jnp.pad(noise.reshape(B * P, 4), ((0, NBP - B * P), (0, 0))).T
    bid = jnp.minimum(jnp.arange(NBP, dtype=jnp.int32) // P, B - 1)
    out = _noised_gt_sc(gt_flat, sc_flat, idx, tt, bid, nz)
    return out.T[: B * P].reshape(B, P, 4)


# speedup vs baseline: 11.0337x; 1.1738x over previous
import functools
import math

import jax
import jax.numpy as jnp
from jax import lax
from jax.experimental import pallas as pl
from jax.experimental.pallas import tpu as pltpu
from jax.experimental.pallas import tpu_sc as plsc

B = 16
G = 64
P = 500
NC = 2
NS = 16
L = 16
NW = NC * NS
NBP = 8192
CH = NBP // NW
NTAB = B * G * 4 + B * 4

HALF_LN_ALPHA = 0.5 * math.log(1.0 - 0.01)

_mesh = plsc.VectorSubcoreMesh(core_axis_name="c", subcore_axis_name="s")


@functools.partial(
    pl.kernel,
    mesh=_mesh,
    compiler_params=pltpu.CompilerParams(needs_layout_passes=False),
    out_type=jax.ShapeDtypeStruct((NW, 4, CH), jnp.float32),
    scratch_types=[
        pltpu.VMEM((NTAB,), jnp.float32),
        pltpu.VMEM((3, CH), jnp.int32),
        pltpu.VMEM((4, CH), jnp.float32),
        pltpu.VMEM((4, CH), jnp.float32),
        pltpu.SemaphoreType.DMA,
        pltpu.SemaphoreType.DMA,
        pltpu.SemaphoreType.DMA,
    ],
)
def _noised_gt_sc(tab_hbm, meta_hbm, nz_hbm, out_hbm,
                  tab_v, meta_v, nz_v, o_v, sem0, sem1, sem2):
    wid = lax.axis_index("s") * NC + lax.axis_index("c")
    cp0 = pltpu.async_copy(tab_hbm, tab_v, sem0)
    cp1 = pltpu.async_copy(meta_hbm.at[wid], meta_v, sem1)
    cp2 = pltpu.async_copy(nz_hbm.at[wid], nz_v, sem2)
    cp0.wait()
    cp1.wait()
    cp2.wait()
    for j in range(CH // L):
        sl = pl.ds(j * L, L)
        li = meta_v[0, sl]
        tf = meta_v[1, sl].astype(jnp.float32)
        b = meta_v[2, sl]
        sqrt_a = jnp.exp(tf * HALF_LN_ALPHA)
        x = 1.0 - sqrt_a * sqrt_a
        y = jnp.where(x > 0.215, 0.681, jnp.where(x > 0.0464, 0.316, 0.1465))
        y = 0.5 * (y + x / y)
        y = 0.5 * (y + x / y)
        y = 0.5 * (y + x / y)
        sqrt_b = jnp.where(x > 0.0, y, 0.0)
        gbase = (b * G + li) * 4
        sbase = B * G * 4 + b * 4
        for c in range(4):
            box = plsc.load_gather(tab_v, [gbase + c])
            s = plsc.load_gather(tab_v, [sbase + c])
            o_v[c, sl] = s * (box * sqrt_a + nz_v[c, sl] * sqrt_b)
    pltpu.sync_copy(o_v, out_hbm.at[wid])


def kernel(gt_boxes, scales, sampled_indices, t, noise):
    tab = jnp.concatenate([gt_boxes.reshape(-1), scales.reshape(-1)])
    pad = NBP - B * P
    idx = jnp.pad(sampled_indices.reshape(-1).astype(jnp.int32), (0, pad))
    tt = jnp.pad(t.reshape(-1).astype(jnp.int32), (0, pad))
    bid = jnp.minimum(jnp.arange(NBP, dtype=jnp.int32) // P, B - 1)
    meta = jnp.stack([idx, tt, bid]).reshape(3, NW, CH).transpose(1, 0, 2)
    nz = (jnp.pad(noise.reshape(B * P, 4), ((0, pad), (0, 0)))
          .reshape(NW, CH, 4).transpose(0, 2, 1))
    out = _noised_gt_sc(tab, meta, nz)
    return out.transpose(0, 2, 1).reshape(NBP, 4)[: B * P].reshape(B, P, 4)
